# Initial kernel scaffold; baseline (speedup 1.0000x reference)
import jax
import jax.numpy as jnp
from jax.experimental import pallas as pl


def kernel(x_game, x_state, edge_attr, Wl, bl, Wr, Wg_s, Wg_d, Wg_e, att_s, att_d, att_e, bg, Wm, bm, ei_gg, ei_ss, ei_hist, ei_in):
    raise NotImplementedError("write your pallas kernel here")


# speedup vs baseline: 3.6932x; 3.6932x over previous
import functools

import jax
import jax.numpy as jnp
from jax import lax
from jax.experimental import pallas as pl
from jax.experimental.pallas import tpu as pltpu
from jax.experimental.pallas import tpu_sc as plsc

N = 10000
E = 320000
D = 128
DE = 16

NC = 2
NS = 16
NW = NC * NS
L = 16

NPAD = 10240
EB = 128
EW = 10240
BPW = EW // EB
EPAD = EW * NW
EROWS = EPAD // EB
DUMP = NPAD - 1

_mesh = plsc.VectorSubcoreMesh(
    core_axis_name="c", subcore_axis_name="s", num_cores=NC, num_subcores=NS)
_sc_params = pltpu.CompilerParams(needs_layout_passes=False)


def _wid():
    return lax.axis_index("c") * NS + lax.axis_index("s")


def _sc_seg_body(x_hbm, srcv_hbm, dstv_hbm, z2d_hbm, z1d_hbm,
                 out_hbm, hist_hbm,
                 src_v, dst_v, rows_v, hist_v, acc_sh):
    c = lax.axis_index("c")
    s = lax.axis_index("s")
    w = c * NS + s
    pltpu.sync_copy(srcv_hbm.at[pl.ds(w * BPW, BPW)], src_v)
    pltpu.sync_copy(dstv_hbm.at[pl.ds(w * BPW, BPW)], dst_v)
    pltpu.sync_copy(z2d_hbm, rows_v)
    pltpu.sync_copy(z1d_hbm, hist_v)
    nblk = NPAD // 128 // NS
    for b in range(nblk):
        pltpu.sync_copy(rows_v, acc_sh.at[pl.ds((s * nblk + b) * 128, 128)])
    plsc.subcore_barrier()

    ones = jnp.full((L,), 1.0, jnp.float32)

    def body(j, carry):
        pltpu.sync_copy(x_hbm.at[src_v.at[j]], rows_v)
        pltpu.sync_copy(rows_v, acc_sh.at[dst_v.at[j]], add=True)
        for k in range(EB // L):
            d16 = dst_v[j, pl.ds(k * L, L)]
            plsc.addupdate_scatter(hist_v, [d16], ones)
        return carry

    lax.fori_loop(0, BPW, body, 0)
    plsc.subcore_barrier()
    for b in range(nblk):
        r0 = (s * nblk + b) * 128
        pltpu.sync_copy(acc_sh.at[pl.ds(r0, 128)], rows_v)
        pltpu.sync_copy(rows_v, out_hbm.at[c, pl.ds(r0, 128)])
    pltpu.sync_copy(hist_v, hist_hbm.at[w])


_sc_seg = pl.kernel(
    _sc_seg_body,
    compiler_params=_sc_params,
    out_type=(
        jax.ShapeDtypeStruct((NC, NPAD, D), jnp.float32),
        jax.ShapeDtypeStruct((NW, NPAD), jnp.float32),
    ),
    mesh=_mesh,
    scratch_types=[
        pltpu.VMEM((BPW, EB), jnp.int32),
        pltpu.VMEM((BPW, EB), jnp.int32),
        pltpu.VMEM((EB, D), jnp.float32),
        pltpu.VMEM((NPAD,), jnp.float32),
        pltpu.VMEM_SHARED((NPAD, D), jnp.float32),
    ],
)


def _sc_gat_logits_body(srcv_hbm, dstv_hbm, eav_hbm, hsa_hbm, hda_hbm,
                        shift_hbm, z1d_hbm,
                        ex_hbm, hist_hbm,
                        src_v, dst_v, ea_v, ex_v, hsa_v, hda_v, hist_v,
                        shift_v):
    c = lax.axis_index("c")
    s = lax.axis_index("s")
    w = c * NS + s
    pltpu.sync_copy(srcv_hbm.at[pl.ds(w * BPW, BPW)], src_v)
    pltpu.sync_copy(dstv_hbm.at[pl.ds(w * BPW, BPW)], dst_v)
    pltpu.sync_copy(eav_hbm.at[pl.ds(w * BPW, BPW)], ea_v)
    pltpu.sync_copy(hsa_hbm, hsa_v)
    pltpu.sync_copy(hda_hbm, hda_v)
    pltpu.sync_copy(z1d_hbm, hist_v)
    pltpu.sync_copy(shift_hbm, shift_v)

    def body(j, carry):
        for k in range(EB // L):
            sl = pl.ds(k * L, L)
            s16 = src_v[j, sl]
            d16 = dst_v[j, sl]
            a = (plsc.load_gather(hsa_v, [s16])
                 + plsc.load_gather(hda_v, [d16])
                 + ea_v[j, sl])
            a = jnp.maximum(a, 0.2 * a)
            ex = jnp.exp(a - shift_v[...])
            ex_v[j, sl] = ex
            plsc.addupdate_scatter(hist_v, [d16], ex)
        return carry

    lax.fori_loop(0, BPW, body, 0)
    pltpu.sync_copy(ex_v, ex_hbm.at[pl.ds(w * BPW, BPW)])
    pltpu.sync_copy(hist_v, hist_hbm.at[w])


_sc_gat_logits = pl.kernel(
    _sc_gat_logits_body,
    compiler_params=_sc_params,
    out_type=(
        jax.ShapeDtypeStruct((EROWS, EB), jnp.float32),
        jax.ShapeDtypeStruct((NW, NPAD), jnp.float32),
    ),
    mesh=_mesh,
    scratch_types=[
        pltpu.VMEM((BPW, EB), jnp.int32),
        pltpu.VMEM((BPW, EB), jnp.int32),
        pltpu.VMEM((BPW, EB), jnp.float32),
        pltpu.VMEM((BPW, EB), jnp.float32),
        pltpu.VMEM((NPAD,), jnp.float32),
        pltpu.VMEM((NPAD,), jnp.float32),
        pltpu.VMEM((NPAD,), jnp.float32),
        pltpu.VMEM((L,), jnp.float32),
    ],
)


def _sc_gat_agg_body(hs_hbm, srcv_hbm, dstv_hbm, exv_hbm, deninv_hbm, z2d_hbm,
                     out_hbm,
                     src_v, dst_v, ex_v, rows_v, dinv_v, alpha_v, acc_sh):
    c = lax.axis_index("c")
    s = lax.axis_index("s")
    w = c * NS + s
    pltpu.sync_copy(srcv_hbm.at[pl.ds(w * BPW, BPW)], src_v)
    pltpu.sync_copy(dstv_hbm.at[pl.ds(w * BPW, BPW)], dst_v)
    pltpu.sync_copy(deninv_hbm, dinv_v)
    pltpu.sync_copy(z2d_hbm, rows_v)
    nblk = NPAD // 128 // NS
    for b in range(nblk):
        pltpu.sync_copy(rows_v, acc_sh.at[pl.ds((s * nblk + b) * 128, 128)])
    plsc.subcore_barrier()

    def body(j, carry):
        pltpu.sync_copy(hs_hbm.at[src_v.at[j]], rows_v)
        pltpu.sync_copy(exv_hbm.at[pl.ds(w * BPW + j, 1)], ex_v)
        for k in range(EB // L):
            sl = pl.ds(k * L, L)
            d16 = dst_v[j, sl]
            alpha_v[sl] = ex_v[0, sl] * plsc.load_gather(dinv_v, [d16])
        for r in range(EB):
            ar = plsc.load_gather(alpha_v, [jnp.full((L,), r, jnp.int32)])
            for m in range(D // L):
                sl = pl.ds(m * L, L)
                rows_v[r, sl] = rows_v[r, sl] * ar
        pltpu.sync_copy(rows_v, acc_sh.at[dst_v.at[j]], add=True)
        return carry

    lax.fori_loop(0, BPW, body, 0)
    plsc.subcore_barrier()
    for b in range(nblk):
        r0 = (s * nblk + b) * 128
        pltpu.sync_copy(acc_sh.at[pl.ds(r0, 128)], rows_v)
        pltpu.sync_copy(rows_v, out_hbm.at[c, pl.ds(r0, 128)])


_sc_gat_agg = pl.kernel(
    _sc_gat_agg_body,
    compiler_params=_sc_params,
    out_type=jax.ShapeDtypeStruct((NC, NPAD, D), jnp.float32),
    mesh=_mesh,
    scratch_types=[
        pltpu.VMEM((BPW, EB), jnp.int32),
        pltpu.VMEM((BPW, EB), jnp.int32),
        pltpu.VMEM((1, EB), jnp.float32),
        pltpu.VMEM((EB, D), jnp.float32),
        pltpu.VMEM((NPAD,), jnp.float32),
        pltpu.VMEM((EB,), jnp.float32),
        pltpu.VMEM_SHARED((NPAD, D), jnp.float32),
    ],
)


BT = 512
NG = NPAD // BT


def _tc_dense_body(p_ref, hist_ref, xd_ref, wl_ref, bl_ref, wr_ref, o_ref):
    deg = jnp.sum(hist_ref[...], axis=0)
    dinv = 1.0 / jnp.maximum(deg, 1.0)
    agg = (p_ref[0] + p_ref[1]) * dinv[:, None]
    o_ref[...] = jax.nn.relu(
        jnp.dot(agg, wl_ref[...], preferred_element_type=jnp.float32)
        + bl_ref[...]
        + jnp.dot(xd_ref[...], wr_ref[...], preferred_element_type=jnp.float32))


def _tc_dense(p, hist, xd, wl, bl, wr):
    return pl.pallas_call(
        _tc_dense_body,
        grid=(NG,),
        in_specs=[
            pl.BlockSpec((NC, BT, D), lambda i: (0, i, 0)),
            pl.BlockSpec((NW, BT), lambda i: (0, i)),
            pl.BlockSpec((BT, D), lambda i: (i, 0)),
            pl.BlockSpec((D, D), lambda i: (0, 0)),
            pl.BlockSpec((1, D), lambda i: (0, 0)),
            pl.BlockSpec((D, D), lambda i: (0, 0)),
        ],
        out_specs=pl.BlockSpec((BT, D), lambda i: (i, 0)),
        out_shape=jax.ShapeDtypeStruct((NPAD, D), jnp.float32),
    )(p, hist, xd, wl, bl.reshape(1, D), wr)


def _tc_final_body(p_ref, hist_ref, ph_ref, bg_ref, wl_ref, bl_ref, wr_ref,
                   wm_ref, bm_ref, o_ref):
    deg = jnp.sum(hist_ref[...], axis=0)
    dinv = 1.0 / jnp.maximum(deg, 1.0)
    agg = (p_ref[0] + p_ref[1]) * dinv[:, None]
    h = jax.nn.relu(ph_ref[0] + ph_ref[1] + bg_ref[...])
    in_x = jax.nn.relu(
        jnp.dot(agg, wl_ref[...], preferred_element_type=jnp.float32)
        + bl_ref[...]
        + jnp.dot(h, wr_ref[...], preferred_element_type=jnp.float32))
    o_ref[...] = (jnp.dot(in_x, wm_ref[...], preferred_element_type=jnp.float32)
                  + bm_ref[...])


def _tc_final(p, hist, ph, bg, wl, bl, wr, wm, bm):
    return pl.pallas_call(
        _tc_final_body,
        grid=(NG,),
        in_specs=[
            pl.BlockSpec((NC, BT, D), lambda i: (0, i, 0)),
            pl.BlockSpec((NW, BT), lambda i: (0, i)),
            pl.BlockSpec((NC, BT, D), lambda i: (0, i, 0)),
            pl.BlockSpec((1, D), lambda i: (0, 0)),
            pl.BlockSpec((D, D), lambda i: (0, 0)),
            pl.BlockSpec((1, D), lambda i: (0, 0)),
            pl.BlockSpec((D, D), lambda i: (0, 0)),
            pl.BlockSpec((D, 1), lambda i: (0, 0)),
            pl.BlockSpec((1, 1), lambda i: (0, 0)),
        ],
        out_specs=pl.BlockSpec((BT, 1), lambda i: (i, 0)),
        out_shape=jax.ShapeDtypeStruct((NPAD, 1), jnp.float32),
    )(p, hist, ph, bg.reshape(1, D), wl, bl.reshape(1, D), wr,
      wm, bm.reshape(1, 1))


def _tc_attn_pre_body(g_ref, st_ref, ws_ref, wd_ref, as_ref, ad_ref,
                      hs_ref, hsa_ref, hda_ref, m_ref):
    i = pl.program_id(0)
    hs = jnp.dot(g_ref[...], ws_ref[...], preferred_element_type=jnp.float32)
    hs_ref[...] = hs
    hsa = jnp.dot(hs, as_ref[...], preferred_element_type=jnp.float32)
    wdv = jnp.dot(wd_ref[...], ad_ref[...], preferred_element_type=jnp.float32)
    hda = jnp.dot(st_ref[...], wdv, preferred_element_type=jnp.float32)
    hsa_ref[...] = hsa
    hda_ref[...] = hda

    @pl.when(i == 0)
    def _():
        m_ref[...] = jnp.full((1, 2), -1e30, jnp.float32)

    cur = jnp.concatenate(
        [jnp.max(hsa).reshape(1, 1), jnp.max(hda).reshape(1, 1)], axis=1)
    m_ref[...] = jnp.maximum(m_ref[...], cur)


def _tc_attn_pre(g, st, ws, wd, att_s, att_d):
    return pl.pallas_call(
        _tc_attn_pre_body,
        grid=(NG,),
        in_specs=[
            pl.BlockSpec((BT, D), lambda i: (i, 0)),
            pl.BlockSpec((BT, D), lambda i: (i, 0)),
            pl.BlockSpec((D, D), lambda i: (0, 0)),
            pl.BlockSpec((D, D), lambda i: (0, 0)),
            pl.BlockSpec((D, 1), lambda i: (0, 0)),
            pl.BlockSpec((D, 1), lambda i: (0, 0)),
        ],
        out_specs=(
            pl.BlockSpec((BT, D), lambda i: (i, 0)),
            pl.BlockSpec((BT, 1), lambda i: (i, 0)),
            pl.BlockSpec((BT, 1), lambda i: (i, 0)),
            pl.BlockSpec((1, 2), lambda i: (0, 0)),
        ),
        out_shape=(
            jax.ShapeDtypeStruct((NPAD, D), jnp.float32),
            jax.ShapeDtypeStruct((NPAD, 1), jnp.float32),
            jax.ShapeDtypeStruct((NPAD, 1), jnp.float32),
            jax.ShapeDtypeStruct((1, 2), jnp.float32),
        ),
    )(g, st, ws, wd, att_s.reshape(D, 1), att_d.reshape(D, 1))


BTE = 2000


def _tc_ea_body(ea_ref, we_ref, o_ref, m_ref):
    i = pl.program_id(0)
    v = jnp.sum(ea_ref[...] * we_ref[...], axis=1, keepdims=True)
    o_ref[...] = v

    @pl.when(i == 0)
    def _():
        m_ref[...] = jnp.full((1, 1), -1e30, jnp.float32)

    m_ref[...] = jnp.maximum(m_ref[...], jnp.max(v).reshape(1, 1))


def _tc_ea(edge_attr, wg_e, att_e):
    we = jnp.dot(wg_e, att_e.reshape(D, 1),
                 preferred_element_type=jnp.float32)
    return pl.pallas_call(
        _tc_ea_body,
        grid=(E // BTE,),
        in_specs=[
            pl.BlockSpec((BTE, DE), lambda i: (i, 0)),
            pl.BlockSpec((1, DE), lambda i: (0, 0)),
        ],
        out_specs=(
            pl.BlockSpec((BTE, 1), lambda i: (i, 0)),
            pl.BlockSpec((1, 1), lambda i: (0, 0)),
        ),
        out_shape=(
            jax.ShapeDtypeStruct((E, 1), jnp.float32),
            jax.ShapeDtypeStruct((1, 1), jnp.float32),
        ),
    )(edge_attr, we.reshape(1, DE))


def _tc_deninv_body(hist_ref, o_ref):
    den = jnp.sum(hist_ref[...], axis=0, keepdims=True)
    o_ref[...] = 1.0 / jnp.maximum(den, 1e-16)


def _tc_deninv(hist):
    return pl.pallas_call(
        _tc_deninv_body,
        out_shape=jax.ShapeDtypeStruct((1, NPAD), jnp.float32),
    )(hist)


def _pad_ei(ei):
    src = jnp.concatenate(
        [ei[0], jnp.zeros((EPAD - E,), ei.dtype)]).astype(jnp.int32)
    dst = jnp.concatenate(
        [ei[1], jnp.full((EPAD - E,), DUMP, ei.dtype)]).astype(jnp.int32)
    return src.reshape(EROWS, EB), dst.reshape(EROWS, EB)


def _pad_x(x):
    return jnp.concatenate([x, jnp.zeros((NPAD - N, D), x.dtype)], axis=0)


def kernel(x_game, x_state, edge_attr, Wl, bl, Wr, Wg_s, Wg_d, Wg_e,
           att_s, att_d, att_e, bg, Wm, bm, ei_gg, ei_ss, ei_hist, ei_in):
    z2d = jnp.zeros((EB, D), jnp.float32)
    z1d = jnp.zeros((NPAD,), jnp.float32)
    sgg, dgg = _pad_ei(ei_gg)
    sss, dss = _pad_ei(ei_ss)
    shh, dhh = _pad_ei(ei_hist)
    sin, din = _pad_ei(ei_in)
    xg = _pad_x(x_game)
    xs = _pad_x(x_state)

    p, hist_gg = _sc_seg(xg, sgg, dgg, z2d, z1d)
    g = _tc_dense(p, hist_gg, xg, Wl[0], bl[0], Wr[0])
    p, _h = _sc_seg(g, sgg, dgg, z2d, z1d)
    g = _tc_dense(p, hist_gg, g, Wl[1], bl[1], Wr[1])

    p, hist_ss = _sc_seg(xs, sss, dss, z2d, z1d)
    st = _tc_dense(p, hist_ss, xs, Wl[2], bl[2], Wr[2])
    p, _h = _sc_seg(st, sss, dss, z2d, z1d)
    st = _tc_dense(p, hist_ss, st, Wl[3], bl[3], Wr[3])

    hs, hsa, hda, m12 = _tc_attn_pre(g, st, Wg_s, Wg_d, att_s, att_d)
    ea, m3 = _tc_ea(edge_attr, Wg_e, att_e)
    shift = jnp.maximum(m12[0, 0] + m12[0, 1] + m3[0, 0], 0.0)
    shift16 = jnp.broadcast_to(shift, (L,))
    eav = jnp.concatenate(
        [ea.reshape(E), jnp.zeros((EPAD - E,), jnp.float32)]).reshape(EROWS, EB)
    ex, hist_den = _sc_gat_logits(shh, dhh, eav, hsa.reshape(NPAD),
                                  hda.reshape(NPAD), shift16, z1d)
    deninv = _tc_deninv(hist_den).reshape(NPAD)
    ph = _sc_gat_agg(hs, shh, dhh, ex, deninv, z2d)

    p, hist_in = _sc_seg(g, sin, din, z2d, z1d)
    out = _tc_final(p, hist_in, ph, bg, Wl[4], bl[4], Wr[4], Wm, bm)
    return out[:N]
